# baseline (device time: 18285 ns/iter reference)
import jax
import jax.numpy as jnp
from jax import lax
from jax.experimental import pallas as pl
from jax.experimental.pallas import tpu as pltpu

MB = 128
NC = 4
CS = MB // NC


def kernel(dy, W):
    m, k = dy.shape
    d, _ = W.shape

    def body(dy_ref, w_ref, out_ref, ypart_ref, yprecv_ref, comm_ref,
             ysem, zsem, xsem):
        my_x = lax.axis_index("x")
        my_y = lax.axis_index("y")
        my_z = lax.axis_index("z")
        y_peer = (my_x, 1 - my_y, my_z)
        z_peer = (my_x, my_y, 1 - my_z)
        x_peer = (1 - my_x, my_y, my_z)

        b = 2 * my_x + my_z
        b_z = 2 * my_x + (1 - my_z)
        b_x = 2 * (1 - my_x) + my_z
        b_d = 2 * (1 - my_x) + (1 - my_z)

        barrier_sem = pltpu.get_barrier_semaphore()
        for peer in (y_peer, z_peer, x_peer):
            pl.semaphore_signal(
                barrier_sem, inc=1,
                device_id=peer, device_id_type=pl.DeviceIdType.MESH,
            )
        pl.semaphore_wait(barrier_sem, 3)

        def y_rdma(j):
            return pltpu.make_async_remote_copy(
                src_ref=ypart_ref.at[pl.ds(j * CS, CS)],
                dst_ref=yprecv_ref.at[pl.ds(j * CS, CS)],
                send_sem=ysem.at[2 * j],
                recv_sem=ysem.at[2 * j + 1],
                device_id=y_peer,
                device_id_type=pl.DeviceIdType.MESH,
            )

        def direct_rdma(c, sems, peer):
            return pltpu.make_async_remote_copy(
                src_ref=comm_ref.at[pl.ds(b * MB + c * CS, CS)],
                dst_ref=comm_ref.at[pl.ds(b * MB + c * CS, CS)],
                send_sem=sems.at[2 * c],
                recv_sem=sems.at[2 * c + 1],
                device_id=peer,
                device_id_type=pl.DeviceIdType.MESH,
            )

        def block_partial(blk, c):
            rows = pl.ds(blk * MB + c * CS, CS)
            out_ref[rows, :] = lax.dot_general(
                dy_ref[rows, :], w_ref[...],
                dimension_numbers=(((1,), (1,)), ((), ())),
                preferred_element_type=jnp.float32,
            )
            return rows

        for c in range(NC):
            rows = block_partial(b, c)
            ypart_ref[pl.ds(c * CS, CS), :] = out_ref[rows, :].astype(
                jnp.bfloat16
            )
            y_rdma(c).start()
        for c in range(NC):
            rows = block_partial(b_d, c)
            ypart_ref[pl.ds(NC * CS + c * CS, CS), :] = out_ref[rows, :].astype(
                jnp.bfloat16
            )
            y_rdma(NC + c).start()

        zs, xs = [], []
        for c in range(NC):
            rows = pl.ds(b * MB + c * CS, CS)
            y_rdma(c).wait_recv()
            acc = out_ref[rows, :] + yprecv_ref[pl.ds(c * CS, CS), :].astype(
                jnp.float32
            )
            out_ref[rows, :] = acc
            comm_ref[rows, :] = acc.astype(jnp.bfloat16)
            z = direct_rdma(c, zsem, z_peer)
            x = direct_rdma(c, xsem, x_peer)
            z.start()
            x.start()
            zs.append(z)
            xs.append(x)

        for c in range(NC):
            rows = pl.ds(b_d * MB + c * CS, CS)
            y_rdma(NC + c).wait_recv()
            out_ref[rows, :] = (
                out_ref[rows, :]
                + yprecv_ref[pl.ds(NC * CS + c * CS, CS), :].astype(jnp.float32)
            )

        for c in range(NC):
            zs[c].wait_recv()
            zrows = pl.ds(b_z * MB + c * CS, CS)
            out_ref[zrows, :] = comm_ref[zrows, :].astype(jnp.float32)
            xs[c].wait_recv()
            xrows = pl.ds(b_x * MB + c * CS, CS)
            out_ref[xrows, :] = comm_ref[xrows, :].astype(jnp.float32)

        for j in range(2 * NC):
            y_rdma(j).wait_send()
        for c in range(NC):
            zs[c].wait_send()
            xs[c].wait_send()

    return pl.pallas_call(
        body,
        out_shape=jax.ShapeDtypeStruct((m, d), jnp.float32),
        in_specs=[
            pl.BlockSpec(memory_space=pltpu.VMEM),
            pl.BlockSpec(memory_space=pltpu.VMEM),
        ],
        out_specs=pl.BlockSpec(memory_space=pltpu.VMEM),
        scratch_shapes=[
            pltpu.VMEM((2 * MB, d), jnp.bfloat16),
            pltpu.VMEM((2 * MB, d), jnp.bfloat16),
            pltpu.VMEM((m, d), jnp.bfloat16),
            pltpu.SemaphoreType.DMA((4 * NC,)),
            pltpu.SemaphoreType.DMA((2 * NC,)),
            pltpu.SemaphoreType.DMA((2 * NC,)),
        ],
        compiler_params=pltpu.CompilerParams(collective_id=0),
    )(dy, W)


# device time: 15181 ns/iter; 1.2045x vs baseline; 1.2045x over previous
import jax
import jax.numpy as jnp
from jax import lax
from jax.experimental import pallas as pl
from jax.experimental.pallas import tpu as pltpu

import os

MB = 128
NC = int(os.environ.get("KERNEL_NC", "2"))
CS = MB // NC


def kernel(dy, W):
    m, k = dy.shape
    d, _ = W.shape

    def body(dy_ref, w_ref, out_ref, ypart_ref, yprecv_ref, comm_ref,
             dyv_ref, wv_ref, csem, ysem, zsem, xsem):
        my_x = lax.axis_index("x")
        my_y = lax.axis_index("y")
        my_z = lax.axis_index("z")
        y_peer = (my_x, 1 - my_y, my_z)
        z_peer = (my_x, my_y, 1 - my_z)
        x_peer = (1 - my_x, my_y, my_z)

        b = 2 * my_x + my_z
        b_z = 2 * my_x + (1 - my_z)
        b_x = 2 * (1 - my_x) + my_z
        b_d = 2 * (1 - my_x) + (1 - my_z)

        barrier_sem = pltpu.get_barrier_semaphore()
        for peer in (y_peer, z_peer, x_peer):
            pl.semaphore_signal(
                barrier_sem, inc=1,
                device_id=peer, device_id_type=pl.DeviceIdType.MESH,
            )
        cp_w = pltpu.make_async_copy(w_ref, wv_ref, csem.at[0])
        cp_b = pltpu.make_async_copy(
            dy_ref.at[pl.ds(b * MB, MB)], dyv_ref.at[pl.ds(0, MB)], csem.at[1]
        )
        cp_d = pltpu.make_async_copy(
            dy_ref.at[pl.ds(b_d * MB, MB)], dyv_ref.at[pl.ds(MB, MB)],
            csem.at[2],
        )
        cp_w.start()
        cp_b.start()
        cp_d.start()
        pl.semaphore_wait(barrier_sem, 3)

        def y_rdma(j):
            return pltpu.make_async_remote_copy(
                src_ref=ypart_ref.at[pl.ds(j * CS, CS)],
                dst_ref=yprecv_ref.at[pl.ds(j * CS, CS)],
                send_sem=ysem.at[2 * j],
                recv_sem=ysem.at[2 * j + 1],
                device_id=y_peer,
                device_id_type=pl.DeviceIdType.MESH,
            )

        def direct_rdma(c, sems, peer):
            return pltpu.make_async_remote_copy(
                src_ref=comm_ref.at[pl.ds(b * MB + c * CS, CS)],
                dst_ref=comm_ref.at[pl.ds(b * MB + c * CS, CS)],
                send_sem=sems.at[2 * c],
                recv_sem=sems.at[2 * c + 1],
                device_id=peer,
                device_id_type=pl.DeviceIdType.MESH,
            )

        def block_partial(blk, voff, c):
            rows = pl.ds(blk * MB + c * CS, CS)
            out_ref[rows, :] = lax.dot_general(
                dyv_ref[pl.ds(voff + c * CS, CS), :], wv_ref[...],
                dimension_numbers=(((1,), (1,)), ((), ())),
                preferred_element_type=jnp.float32,
            )
            return rows

        cp_w.wait()
        cp_b.wait()
        for c in range(NC):
            rows = block_partial(b, 0, c)
            ypart_ref[pl.ds(c * CS, CS), :] = out_ref[rows, :].astype(
                jnp.bfloat16
            )
            y_rdma(c).start()
        cp_d.wait()
        for c in range(NC):
            rows = block_partial(b_d, MB, c)
            ypart_ref[pl.ds(NC * CS + c * CS, CS), :] = out_ref[rows, :].astype(
                jnp.bfloat16
            )
            y_rdma(NC + c).start()

        zs, xs = [], []
        for c in range(NC):
            rows = pl.ds(b * MB + c * CS, CS)
            y_rdma(c).wait_recv()
            acc = out_ref[rows, :] + yprecv_ref[pl.ds(c * CS, CS), :].astype(
                jnp.float32
            )
            out_ref[rows, :] = acc
            comm_ref[rows, :] = acc.astype(jnp.bfloat16)
            z = direct_rdma(c, zsem, z_peer)
            x = direct_rdma(c, xsem, x_peer)
            z.start()
            x.start()
            zs.append(z)
            xs.append(x)

        for c in range(NC):
            rows = pl.ds(b_d * MB + c * CS, CS)
            y_rdma(NC + c).wait_recv()
            out_ref[rows, :] = (
                out_ref[rows, :]
                + yprecv_ref[pl.ds(NC * CS + c * CS, CS), :].astype(jnp.float32)
            )

        for c in range(NC):
            zs[c].wait_recv()
            zrows = pl.ds(b_z * MB + c * CS, CS)
            out_ref[zrows, :] = comm_ref[zrows, :].astype(jnp.float32)
            xs[c].wait_recv()
            xrows = pl.ds(b_x * MB + c * CS, CS)
            out_ref[xrows, :] = comm_ref[xrows, :].astype(jnp.float32)

        for j in range(2 * NC):
            y_rdma(j).wait_send()
        for c in range(NC):
            zs[c].wait_send()
            xs[c].wait_send()

    return pl.pallas_call(
        body,
        out_shape=jax.ShapeDtypeStruct((m, d), jnp.float32),
        in_specs=[
            pl.BlockSpec(memory_space=pltpu.MemorySpace.HBM),
            pl.BlockSpec(memory_space=pltpu.MemorySpace.HBM),
        ],
        out_specs=pl.BlockSpec(memory_space=pltpu.VMEM),
        scratch_shapes=[
            pltpu.VMEM((2 * MB, d), jnp.bfloat16),
            pltpu.VMEM((2 * MB, d), jnp.bfloat16),
            pltpu.VMEM((m, d), jnp.bfloat16),
            pltpu.VMEM((2 * MB, k), jnp.float32),
            pltpu.VMEM((d, k), jnp.float32),
            pltpu.SemaphoreType.DMA((3,)),
            pltpu.SemaphoreType.DMA((4 * NC,)),
            pltpu.SemaphoreType.DMA((2 * NC,)),
            pltpu.SemaphoreType.DMA((2 * NC,)),
        ],
        compiler_params=pltpu.CompilerParams(collective_id=0),
    )(dy, W)
